# baseline (speedup 1.0000x reference)
import functools

import jax
import jax.numpy as jnp
from jax import lax
from jax.experimental import pallas as pl
from jax.experimental.pallas import tpu as pltpu
from jax.experimental.pallas import tpu_sc as plsc

VOCAB = 3
EMBED = 512
SL = EMBED // 128
ROWS = 4 * 8192
NUM_CORES = 2
NUM_SUBCORES = 16
NW = NUM_CORES * NUM_SUBCORES
R_PER_W = ROWS // NW
CHUNK = 64
NCHUNK = R_PER_W // CHUNK

_mesh = plsc.VectorSubcoreMesh(core_axis_name="c", subcore_axis_name="s")


@functools.partial(
    pl.kernel,
    mesh=_mesh,
    out_type=jax.ShapeDtypeStruct((ROWS, SL, 128), jnp.float32),
    scratch_types=[
        pltpu.VMEM((R_PER_W,), jnp.int32),
        pltpu.VMEM((CHUNK, SL, 128), jnp.float32),
        pltpu.VMEM((CHUNK, SL, 128), jnp.float32),
        pltpu.SemaphoreType.DMA,
        pltpu.SemaphoreType.DMA,
    ],
)
def _embed_sc(idx_hbm, w_hbm, out_hbm, idx_v, buf0, buf1, gsem, ssem):
    sid = lax.axis_index("s")
    wid = sid * NUM_CORES + lax.axis_index("c")
    base = wid * R_PER_W

    pltpu.sync_copy(idx_hbm.at[pl.ds(base, R_PER_W)], idx_v)

    bufs = (buf0, buf1)

    def gather(c):
        return pltpu.async_copy(
            w_hbm.at[idx_v.at[pl.ds(c * CHUNK, CHUNK)]], bufs[c % 2], gsem
        )

    def scatter(c):
        return pltpu.async_copy(
            bufs[c % 2], out_hbm.at[pl.ds(base + c * CHUNK, CHUNK)], ssem
        )

    g = {}
    s = {}
    for c in range(NCHUNK):
        if c >= 2:
            s[c - 2].wait()
        g[c] = gather(c)
        if c >= 1:
            g[c - 1].wait()
            s[c - 1] = scatter(c - 1)
    g[NCHUNK - 1].wait()
    s[NCHUNK - 1] = scatter(NCHUNK - 1)
    s[NCHUNK - 2].wait()
    s[NCHUNK - 1].wait()


def kernel(input, weight):
    idx = input.reshape(-1).astype(jnp.int32)
    w3 = weight.reshape(VOCAB, SL, 128)
    out = _embed_sc(idx, w3)
    return out.reshape(input.shape + (EMBED,))


# speedup vs baseline: 4.3649x; 4.3649x over previous
import functools

import jax
import jax.numpy as jnp
from jax import lax
from jax.experimental import pallas as pl
from jax.experimental.pallas import tpu as pltpu
from jax.experimental.pallas import tpu_sc as plsc

VOCAB = 3
EMBED = 512
LANES = 16
ROWS = 4 * 8192
NUM_CORES = 2
NUM_SUBCORES = 16
NW = NUM_CORES * NUM_SUBCORES
R_PER_W = ROWS // NW
CHUNK = 64
NCHUNK = R_PER_W // CHUNK

_mesh = plsc.VectorSubcoreMesh(core_axis_name="c", subcore_axis_name="s")


@functools.partial(
    pl.kernel,
    mesh=_mesh,
    out_type=jax.ShapeDtypeStruct((ROWS, EMBED), jnp.float32),
    scratch_types=[
        pltpu.VMEM((R_PER_W,), jnp.int32),
        pltpu.VMEM((VOCAB, EMBED), jnp.float32),
        pltpu.VMEM((CHUNK, EMBED), jnp.float32),
        pltpu.VMEM((CHUNK, EMBED), jnp.float32),
        pltpu.VMEM_SHARED((NUM_SUBCORES, R_PER_W), jnp.int32),
        pltpu.SMEM((CHUNK,), jnp.int32),
        pltpu.SemaphoreType.DMA,
    ],
)
def _embed_sc(idx_hbm, w_hbm, out_hbm, idx_v, tbl_v, buf0, buf1, idx_sh,
              idx_sm, ssem):
    sid = lax.axis_index("s")
    wid = sid * NUM_CORES + lax.axis_index("c")
    base = wid * R_PER_W

    pltpu.sync_copy(w_hbm, tbl_v)
    pltpu.sync_copy(idx_hbm.at[pl.ds(base, R_PER_W)], idx_v)
    pltpu.sync_copy(idx_v, idx_sh.at[sid])

    bufs = (buf0, buf1)

    def fill(c, buf):
        pltpu.sync_copy(idx_sh.at[sid, pl.ds(c * CHUNK, CHUNK)], idx_sm)

        def row(i, carry):
            r = idx_sm[i]
            for k in range(EMBED // LANES):
                buf[i, pl.ds(k * LANES, LANES)] = tbl_v[r, pl.ds(k * LANES, LANES)]
            return carry
        lax.fori_loop(0, CHUNK, row, 0)

    def scatter(c):
        return pltpu.async_copy(
            bufs[c % 2], out_hbm.at[pl.ds(base + c * CHUNK, CHUNK)], ssem
        )

    s = {}
    for c in range(NCHUNK):
        if c >= 2:
            s[c - 2].wait()
        fill(c, bufs[c % 2])
        s[c] = scatter(c)
    s[NCHUNK - 2].wait()
    s[NCHUNK - 1].wait()


def kernel(input, weight):
    idx = input.reshape(-1).astype(jnp.int32)
    out = _embed_sc(idx, weight)
    return out.reshape(input.shape + (EMBED,))


# speedup vs baseline: 9.9274x; 2.2744x over previous
import functools

import jax
import jax.numpy as jnp
from jax import lax
from jax.experimental import pallas as pl
from jax.experimental.pallas import tpu as pltpu
from jax.experimental.pallas import tpu_sc as plsc

VOCAB = 3
EMBED = 512
LANES = 16
ROWS = 4 * 8192
NUM_CORES = 2
NUM_SUBCORES = 16
NW = NUM_CORES * NUM_SUBCORES
R_PER_W = ROWS // NW
CHUNK = 64
NCHUNK = R_PER_W // CHUNK

_mesh = plsc.VectorSubcoreMesh(core_axis_name="c", subcore_axis_name="s")


@functools.partial(
    pl.kernel,
    mesh=_mesh,
    out_type=jax.ShapeDtypeStruct((ROWS, EMBED), jnp.float32),
    scratch_types=[
        pltpu.VMEM((R_PER_W,), jnp.int32),
        pltpu.VMEM((VOCAB, EMBED), jnp.float32),
        pltpu.VMEM((CHUNK, EMBED), jnp.float32),
        pltpu.VMEM((CHUNK, EMBED), jnp.float32),
        pltpu.VMEM_SHARED((NUM_SUBCORES, R_PER_W), jnp.int32),
        pltpu.SMEM((CHUNK,), jnp.int32),
        pltpu.SemaphoreType.DMA,
    ],
)
def _embed_sc(idx_hbm, w_hbm, out_hbm, idx_v, tbl_v, buf0, buf1, idx_sh,
              idx_sm, ssem):
    sid = lax.axis_index("s")
    wid = sid * NUM_CORES + lax.axis_index("c")
    base = wid * R_PER_W

    pltpu.sync_copy(w_hbm, tbl_v)
    pltpu.sync_copy(idx_hbm.at[pl.ds(base, R_PER_W)], idx_v)
    pltpu.sync_copy(idx_v, idx_sh.at[sid])

    bufs = (buf0, buf1)

    def fill(c, buf):
        pltpu.sync_copy(idx_sh.at[sid, pl.ds(c * CHUNK, CHUNK)], idx_sm)

        @plsc.parallel_loop(0, CHUNK, 1, unroll=4)
        def row(i):
            r = idx_sm[i]
            for k in range(EMBED // LANES):
                buf[i, pl.ds(k * LANES, LANES)] = tbl_v[r, pl.ds(k * LANES, LANES)]

    def wait_one_scatter(b):
        pltpu.make_async_copy(
            bufs[b], out_hbm.at[pl.ds(base, CHUNK)], ssem
        ).wait()

    def outer(g, carry):
        for b in range(2):
            c = g * 2 + b

            @pl.when(c >= 2)
            def _():
                wait_one_scatter(b)

            fill(c, bufs[b])
            pltpu.async_copy(
                bufs[b], out_hbm.at[pl.ds(base + c * CHUNK, CHUNK)], ssem
            )
        return carry

    lax.fori_loop(0, NCHUNK // 2, outer, 0)
    wait_one_scatter(0)
    wait_one_scatter(1)


def kernel(input, weight):
    idx = input.reshape(-1).astype(jnp.int32)
    out = _embed_sc(idx, weight)
    return out.reshape(input.shape + (EMBED,))


# speedup vs baseline: 13.0691x; 1.3165x over previous
import functools

import jax
import jax.numpy as jnp
from jax import lax
from jax.experimental import pallas as pl
from jax.experimental.pallas import tpu as pltpu
from jax.experimental.pallas import tpu_sc as plsc

VOCAB = 3
EMBED = 512
LANES = 16
ROWS = 4 * 8192
NUM_CORES = 2
NUM_SUBCORES = 16
NW = NUM_CORES * NUM_SUBCORES
R_PER_W = ROWS // NW
CHUNK = 64
NCHUNK = R_PER_W // CHUNK

_mesh = plsc.VectorSubcoreMesh(core_axis_name="c", subcore_axis_name="s")


@functools.partial(
    pl.kernel,
    mesh=_mesh,
    out_type=jax.ShapeDtypeStruct((ROWS, EMBED), jnp.float32),
    scratch_types=[
        pltpu.VMEM((R_PER_W,), jnp.int32),
        pltpu.VMEM((VOCAB, EMBED), jnp.float32),
        pltpu.VMEM((CHUNK, EMBED), jnp.float32),
        pltpu.VMEM((CHUNK, EMBED), jnp.float32),
        pltpu.VMEM_SHARED((NUM_SUBCORES, R_PER_W), jnp.int32),
        pltpu.SMEM((CHUNK,), jnp.int32),
        pltpu.SemaphoreType.DMA,
    ],
)
def _embed_sc(idx_hbm, w_hbm, out_hbm, idx_v, tbl_v, buf0, buf1, idx_sh,
              idx_sm, ssem):
    sid = lax.axis_index("s")
    wid = sid * NUM_CORES + lax.axis_index("c")
    base = wid * R_PER_W

    pltpu.sync_copy(w_hbm, tbl_v)
    pltpu.sync_copy(idx_hbm.at[pl.ds(base, R_PER_W)], idx_v)
    pltpu.sync_copy(idx_v, idx_sh.at[sid])

    bufs = (buf0, buf1)

    def fill(c, buf):
        pltpu.sync_copy(idx_sh.at[sid, pl.ds(c * CHUNK, CHUNK)], idx_sm)

        @plsc.parallel_loop(0, CHUNK, 1, unroll=8)
        def row(i):
            r = idx_sm[i]
            for k in range(EMBED // LANES):
                buf[i, pl.ds(k * LANES, LANES)] = tbl_v[r, pl.ds(k * LANES, LANES)]

    def wait_one_scatter(b):
        pltpu.make_async_copy(
            bufs[b], out_hbm.at[pl.ds(base, CHUNK)], ssem
        ).wait()

    def outer(g, carry):
        for b in range(2):
            c = g * 2 + b

            @pl.when(c >= 2)
            def _():
                wait_one_scatter(b)

            fill(c, bufs[b])
            pltpu.async_copy(
                bufs[b], out_hbm.at[pl.ds(base + c * CHUNK, CHUNK)], ssem
            )
        return carry

    lax.fori_loop(0, NCHUNK // 2, outer, 0)
    wait_one_scatter(0)
    wait_one_scatter(1)


def kernel(input, weight):
    idx = input.reshape(-1).astype(jnp.int32)
    out = _embed_sc(idx, weight)
    return out.reshape(input.shape + (EMBED,))
